# Initial kernel scaffold; baseline (speedup 1.0000x reference)
import jax
import jax.numpy as jnp
from jax.experimental import pallas as pl


def kernel(items, samples, edge_index, emb_weight, W, b):
    raise NotImplementedError("write your pallas kernel here")


# speedup vs baseline: 20.0330x; 20.0330x over previous
import functools

import jax
import jax.numpy as jnp
from jax import lax
from jax.experimental import pallas as pl
from jax.experimental.pallas import tpu as pltpu
from jax.experimental.pallas import tpu_sc as plsc

N_NODES = 10000
D = 128
NP = 10240
E = 320000
B = 4096
S = 20
NC, NS = 2, 16
NW = NC * NS
CHUNK = 128
EPW_CH = 80
EP = NW * EPW_CH * CHUNK
IDX_BLK = 8
G_TOT = B * (S + 1)
GPW_CH = 24
G_PAD = NW * GPW_CH * CHUNK

_mesh = plsc.VectorSubcoreMesh(core_axis_name="c", subcore_axis_name="s")


@functools.partial(
    pl.kernel,
    out_type=jax.ShapeDtypeStruct((NC * NP,), jnp.float32),
    mesh=_mesh,
    scratch_types=[
        pltpu.VMEM((IDX_BLK, CHUNK), jnp.int32),
        pltpu.VMEM((CHUNK,), jnp.float32),
        pltpu.VMEM_SHARED((NP,), jnp.float32),
    ],
)
def _deg_kernel(dst2d_hbm, zvec_hbm, deg_hbm, idx_v, ones_v, deg_sh):
    cid = lax.axis_index("c")
    sid = lax.axis_index("s")
    wid = sid * NC + cid
    for j in range(CHUNK // 16):
        ones_v[pl.ds(j * 16, 16)] = jnp.full((16,), 1.0, jnp.float32)
    rows = NP // NS
    pltpu.sync_copy(zvec_hbm.at[pl.ds(sid * rows, rows)],
                    deg_sh.at[pl.ds(sid * rows, rows)])
    plsc.subcore_barrier()

    rbase = wid * EPW_CH

    def outer(ob, carry):
        row0 = rbase + ob * IDX_BLK
        pltpu.sync_copy(dst2d_hbm.at[pl.ds(row0, IDX_BLK)], idx_v)
        for j in range(IDX_BLK):
            pltpu.sync_copy(ones_v, deg_sh.at[idx_v.at[j]], add=True)
        return carry

    lax.fori_loop(0, EPW_CH // IDX_BLK, outer, 0)
    plsc.subcore_barrier()
    pltpu.sync_copy(deg_sh.at[pl.ds(sid * rows, rows)],
                    deg_hbm.at[pl.ds(cid * NP + sid * rows, rows)])


@functools.partial(
    pl.kernel,
    out_type=jax.ShapeDtypeStruct((NC * NP, D), jnp.float32),
    mesh=_mesh,
    scratch_types=[
        pltpu.VMEM((IDX_BLK, CHUNK), jnp.int32),
        pltpu.VMEM((IDX_BLK, CHUNK), jnp.int32),
        pltpu.VMEM((CHUNK, D), jnp.float32),
        pltpu.VMEM((CHUNK, D), jnp.float32),
        pltpu.VMEM_SHARED((NP, D), jnp.float32),
        pltpu.SemaphoreType.DMA,
        pltpu.SemaphoreType.DMA,
    ],
)
def _scatter_kernel(g_hbm, src2d_hbm, dst2d_hbm, zrows_hbm, acc_hbm,
                    sidx_v, didx_v, rows0_v, rows1_v, acc_sh, sem0, sem1):
    cid = lax.axis_index("c")
    sid = lax.axis_index("s")
    wid = sid * NC + cid
    for r in range(NP // CHUNK // NS):
        row0 = (sid * (NP // CHUNK // NS) + r) * CHUNK
        pltpu.sync_copy(zrows_hbm, acc_sh.at[pl.ds(row0, CHUNK)])
    plsc.subcore_barrier()

    rbase = wid * EPW_CH
    row_bufs = (rows0_v, rows1_v)
    sems = (sem0, sem1)

    def outer(ob, carry):
        row0 = rbase + ob * IDX_BLK
        pltpu.sync_copy(src2d_hbm.at[pl.ds(row0, IDX_BLK)], sidx_v)
        pltpu.sync_copy(dst2d_hbm.at[pl.ds(row0, IDX_BLK)], didx_v)
        desc = pltpu.async_copy(g_hbm.at[sidx_v.at[0]], row_bufs[0], sems[0])
        for j in range(IDX_BLK):
            nxt = None
            if j + 1 < IDX_BLK:
                nxt = pltpu.async_copy(g_hbm.at[sidx_v.at[j + 1]],
                                       row_bufs[(j + 1) % 2], sems[(j + 1) % 2])
            desc.wait()
            pltpu.sync_copy(row_bufs[j % 2], acc_sh.at[didx_v.at[j]], add=True)
            desc = nxt
        return carry

    lax.fori_loop(0, EPW_CH // IDX_BLK, outer, 0)
    plsc.subcore_barrier()
    for r in range(NP // CHUNK // NS):
        row0 = (sid * (NP // CHUNK // NS) + r) * CHUNK
        pltpu.sync_copy(acc_sh.at[pl.ds(row0, CHUNK)],
                        acc_hbm.at[pl.ds(cid * NP + row0, CHUNK)])


@functools.partial(
    pl.kernel,
    out_type=jax.ShapeDtypeStruct((G_PAD, D), jnp.float32),
    mesh=_mesh,
    scratch_types=[
        pltpu.VMEM((GPW_CH, CHUNK), jnp.int32),
        pltpu.VMEM((CHUNK, D), jnp.float32),
        pltpu.VMEM((CHUNK, D), jnp.float32),
        pltpu.SemaphoreType.DMA,
        pltpu.SemaphoreType.DMA,
    ],
)
def _gather_kernel(upd_hbm, gidx2d_hbm, out_hbm, idx_v, rows0_v, rows1_v,
                   sem0, sem1):
    cid = lax.axis_index("c")
    sid = lax.axis_index("s")
    wid = sid * NC + cid
    rbase = wid * GPW_CH
    pltpu.sync_copy(gidx2d_hbm.at[pl.ds(rbase, GPW_CH)], idx_v)
    row_bufs = (rows0_v, rows1_v)
    sems = (sem0, sem1)
    desc = pltpu.async_copy(upd_hbm.at[idx_v.at[0]], row_bufs[0], sems[0])
    for j in range(GPW_CH):
        nxt = None
        if j + 1 < GPW_CH:
            nxt = pltpu.async_copy(upd_hbm.at[idx_v.at[j + 1]],
                                   row_bufs[(j + 1) % 2], sems[(j + 1) % 2])
        desc.wait()
        pltpu.sync_copy(row_bufs[j % 2],
                        out_hbm.at[pl.ds((rbase + j) * CHUNK, CHUNK)])
        desc = nxt


_RB = 2048


def _tc1_body(emb_ref, w_ref, deg_ref, g_ref, dinv_ref):
    deg = deg_ref[0] + deg_ref[1] + 1.0
    dinv = lax.rsqrt(deg)
    dinv_ref[...] = dinv
    h = jnp.dot(emb_ref[...], w_ref[...], preferred_element_type=jnp.float32)
    g_ref[...] = h * dinv


_tc1 = pl.pallas_call(
    _tc1_body,
    grid=(NP // _RB,),
    in_specs=[
        pl.BlockSpec((_RB, D), lambda i: (i, 0)),
        pl.BlockSpec((D, D), lambda i: (0, 0)),
        pl.BlockSpec((NC, _RB, 1), lambda i: (0, i, 0)),
    ],
    out_specs=[
        pl.BlockSpec((_RB, D), lambda i: (i, 0)),
        pl.BlockSpec((_RB, 1), lambda i: (i, 0)),
    ],
    out_shape=[
        jax.ShapeDtypeStruct((NP, D), jnp.float32),
        jax.ShapeDtypeStruct((NP, 1), jnp.float32),
    ],
)


def _tc2_body(acc_ref, g_ref, dinv_ref, b_ref, upd_ref):
    upd_ref[...] = (dinv_ref[...] * (acc_ref[0] + acc_ref[1] + g_ref[...])
                    + b_ref[...])


_tc2 = pl.pallas_call(
    _tc2_body,
    grid=(NP // _RB,),
    in_specs=[
        pl.BlockSpec((NC, _RB, D), lambda i: (0, i, 0)),
        pl.BlockSpec((_RB, D), lambda i: (i, 0)),
        pl.BlockSpec((_RB, 1), lambda i: (i, 0)),
        pl.BlockSpec((1, D), lambda i: (0, 0)),
    ],
    out_specs=pl.BlockSpec((_RB, D), lambda i: (i, 0)),
    out_shape=jax.ShapeDtypeStruct((NP, D), jnp.float32),
)

_RB3 = 512


def _tc3_body(it_ref, sm_ref, out_ref):
    it = it_ref[...]
    sm = sm_ref[...]
    out_ref[...] = jnp.sum(it[:, None, :] * sm, axis=-1)


_tc3 = pl.pallas_call(
    _tc3_body,
    grid=(B // _RB3,),
    in_specs=[
        pl.BlockSpec((_RB3, D), lambda i: (i, 0)),
        pl.BlockSpec((_RB3, S, D), lambda i: (i, 0, 0)),
    ],
    out_specs=pl.BlockSpec((_RB3, S), lambda i: (i, 0)),
    out_shape=jax.ShapeDtypeStruct((B, S), jnp.float32),
)


def kernel(items, samples, edge_index, emb_weight, W, b):
    f32 = jnp.float32
    i32 = jnp.int32
    src = edge_index[0].astype(i32)
    dst = edge_index[1].astype(i32)
    npad = EP - E
    pad = (jnp.arange(npad, dtype=i32) % (NP - N_NODES)) + N_NODES
    src2d = jnp.concatenate([src, pad]).reshape(EP // CHUNK, CHUNK)
    dst2d = jnp.concatenate([dst, pad]).reshape(EP // CHUNK, CHUNK)
    emb_p = jnp.pad(emb_weight.astype(f32), ((0, NP - N_NODES), (0, 0)))

    zvec = jnp.zeros((NP,), f32)
    zrows = jnp.zeros((CHUNK, D), f32)

    deg = _deg_kernel(dst2d, zvec)
    g, dinv = _tc1(emb_p, W.astype(f32), deg.reshape(NC, NP, 1))
    acc = _scatter_kernel(g, src2d, dst2d, zrows)
    upd = _tc2(acc.reshape(NC, NP, D), g, dinv, b.astype(f32).reshape(1, D))

    gpad = (jnp.arange(G_PAD - G_TOT, dtype=i32) % (NP - N_NODES)) + N_NODES
    gidx2d = jnp.concatenate(
        [items.astype(i32), samples.astype(i32).reshape(-1), gpad]
    ).reshape(G_PAD // CHUNK, CHUNK)
    rows = _gather_kernel(upd, gidx2d)
    return _tc3(rows[:B], rows[B:G_TOT].reshape(B, S, D))
